# baseline (speedup 1.0000x reference)
import functools

import jax
import jax.numpy as jnp
from jax.experimental import pallas as pl
from jax.experimental.pallas import tpu as pltpu

B, H, S, D = 1, 16, 2048, 128
BQ = 2048
SUB = 256


def _sub_attn(q, kb, vb, maxk, scale):
    qs = q * scale
    nq = jnp.sqrt(jnp.sum(qs * qs, axis=1, keepdims=True))
    m = nq * maxk * 1.01 + 0.1
    qb = qs.astype(jnp.bfloat16)
    s = jax.lax.dot_general(qb, kb, (((1,), (1,)), ((), ())),
                            preferred_element_type=jnp.float32)
    p = jnp.exp2(s - m).astype(jnp.bfloat16)
    l = jnp.sum(p, axis=1, keepdims=True).astype(jnp.float32) + 1e-30
    o = jax.lax.dot_general(p, vb,
                            (((1,), (0,)), ((), ())),
                            preferred_element_type=jnp.float32)
    return o * (1.0 / l)


def _attn_block(q_ref, k_ref, v_ref, o_ref, kb_ref, vb_ref, *, scale):
    kf = k_ref[0]
    kb_ref[...] = kf.astype(jnp.bfloat16)
    vb_ref[...] = v_ref[0].astype(jnp.bfloat16)
    maxk = jnp.sqrt(jnp.max(jnp.sum(kf * kf, axis=1)))

    kb = kb_ref[...]
    vb = vb_ref[...]
    for j in range(BQ // SUB):
        o_ref[0, j * SUB:(j + 1) * SUB] = _sub_attn(
            q_ref[0, j * SUB:(j + 1) * SUB], kb, vb, maxk, scale)


def kernel(query, key, value):
    scale = D ** (-0.5) * 1.4426950408889634
    q = query.reshape(H, S, D)
    k = key.reshape(H, S, D)
    v = value.reshape(H, S, D)
    out = pl.pallas_call(
        functools.partial(_attn_block, scale=scale),
        grid=(H,),
        in_specs=[
            pl.BlockSpec((1, BQ, D), lambda h: (h, 0, 0)),
            pl.BlockSpec((1, S, D), lambda h: (h, 0, 0)),
            pl.BlockSpec((1, S, D), lambda h: (h, 0, 0)),
        ],
        out_specs=pl.BlockSpec((1, BQ, D), lambda h: (h, 0, 0)),
        out_shape=jax.ShapeDtypeStruct((H, S, D), jnp.float32),
        scratch_shapes=[
            pltpu.VMEM((S, D), jnp.bfloat16),
            pltpu.VMEM((S, D), jnp.bfloat16),
        ],
        compiler_params=pltpu.CompilerParams(
            dimension_semantics=("parallel",),
        ),
    )(q, k, v)
    return out.reshape(B, H, S, D)


# speedup vs baseline: 1.0776x; 1.0776x over previous
import functools

import jax
import jax.numpy as jnp
from jax.experimental import pallas as pl
from jax.experimental.pallas import tpu as pltpu

B, H, S, D = 1, 16, 2048, 128
HPB = 2
BQ = 2048
SUB = 512


def _sub_attn(q, kb, vb, maxk, scale):
    qs = q * scale
    nq = jnp.sqrt(jnp.sum(qs * qs, axis=1, keepdims=True))
    m = nq * maxk * 1.01 + 0.1
    qb = qs.astype(jnp.bfloat16)
    s = jax.lax.dot_general(qb, kb, (((1,), (1,)), ((), ())),
                            preferred_element_type=jnp.float32)
    p = jnp.exp2(s - m).astype(jnp.bfloat16)
    l = jnp.sum(p, axis=1, keepdims=True).astype(jnp.float32) + 1e-30
    o = jax.lax.dot_general(p, vb,
                            (((1,), (0,)), ((), ())),
                            preferred_element_type=jnp.float32)
    return o * (1.0 / l)


def _attn_block(q_ref, k_ref, v_ref, o_ref, kb_ref, vb_ref, *, scale):
    for g in range(HPB):
        kf = k_ref[g]
        kb_ref[g] = kf.astype(jnp.bfloat16)
        vb_ref[g] = v_ref[g].astype(jnp.bfloat16)
        maxk = jnp.sqrt(jnp.max(jnp.sum(kf * kf, axis=1)))
        kb = kb_ref[g]
        vb = vb_ref[g]
        for j in range(BQ // SUB):
            o_ref[g, j * SUB:(j + 1) * SUB] = _sub_attn(
                q_ref[g, j * SUB:(j + 1) * SUB], kb, vb, maxk, scale)


def kernel(query, key, value):
    scale = D ** (-0.5) * 1.4426950408889634
    q = query.reshape(H, S, D)
    k = key.reshape(H, S, D)
    v = value.reshape(H, S, D)
    out = pl.pallas_call(
        functools.partial(_attn_block, scale=scale),
        grid=(H // HPB,),
        in_specs=[
            pl.BlockSpec((HPB, BQ, D), lambda h: (h, 0, 0)),
            pl.BlockSpec((HPB, S, D), lambda h: (h, 0, 0)),
            pl.BlockSpec((HPB, S, D), lambda h: (h, 0, 0)),
        ],
        out_specs=pl.BlockSpec((HPB, BQ, D), lambda h: (h, 0, 0)),
        out_shape=jax.ShapeDtypeStruct((H, S, D), jnp.float32),
        scratch_shapes=[
            pltpu.VMEM((HPB, S, D), jnp.bfloat16),
            pltpu.VMEM((HPB, S, D), jnp.bfloat16),
        ],
        compiler_params=pltpu.CompilerParams(
            dimension_semantics=("parallel",),
        ),
    )(q, k, v)
    return out.reshape(B, H, S, D)


# speedup vs baseline: 1.0946x; 1.0157x over previous
import functools

import jax
import jax.numpy as jnp
from jax.experimental import pallas as pl
from jax.experimental.pallas import tpu as pltpu

B, H, S, D = 1, 16, 2048, 128
HPB = 4
BQ = 2048
SUB = 512


def _sub_attn(q, kb, vb, maxk, scale):
    qs = q * scale
    nq = jnp.sqrt(jnp.sum(qs * qs, axis=1, keepdims=True))
    m = nq * maxk * 1.01 + 0.1
    qb = qs.astype(jnp.bfloat16)
    s = jax.lax.dot_general(qb, kb, (((1,), (1,)), ((), ())),
                            preferred_element_type=jnp.float32)
    p = jnp.exp2(s - m).astype(jnp.bfloat16)
    l = jnp.sum(p, axis=1, keepdims=True).astype(jnp.float32) + 1e-30
    o = jax.lax.dot_general(p, vb,
                            (((1,), (0,)), ((), ())),
                            preferred_element_type=jnp.float32)
    return o * (1.0 / l)


def _attn_block(q_ref, k_ref, v_ref, o_ref, kb_ref, vb_ref, *, scale):
    for g in range(HPB):
        kf = k_ref[g]
        kb_ref[g] = kf.astype(jnp.bfloat16)
        vb_ref[g] = v_ref[g].astype(jnp.bfloat16)
        maxk = jnp.sqrt(jnp.max(jnp.sum(kf * kf, axis=1)))
        kb = kb_ref[g]
        vb = vb_ref[g]
        for j in range(BQ // SUB):
            o_ref[g, j * SUB:(j + 1) * SUB] = _sub_attn(
                q_ref[g, j * SUB:(j + 1) * SUB], kb, vb, maxk, scale)


def kernel(query, key, value):
    scale = D ** (-0.5) * 1.4426950408889634
    q = query.reshape(H, S, D)
    k = key.reshape(H, S, D)
    v = value.reshape(H, S, D)
    out = pl.pallas_call(
        functools.partial(_attn_block, scale=scale),
        grid=(H // HPB,),
        in_specs=[
            pl.BlockSpec((HPB, BQ, D), lambda h: (h, 0, 0)),
            pl.BlockSpec((HPB, S, D), lambda h: (h, 0, 0)),
            pl.BlockSpec((HPB, S, D), lambda h: (h, 0, 0)),
        ],
        out_specs=pl.BlockSpec((HPB, BQ, D), lambda h: (h, 0, 0)),
        out_shape=jax.ShapeDtypeStruct((H, S, D), jnp.float32),
        scratch_shapes=[
            pltpu.VMEM((HPB, S, D), jnp.bfloat16),
            pltpu.VMEM((HPB, S, D), jnp.bfloat16),
        ],
        compiler_params=pltpu.CompilerParams(
            dimension_semantics=("parallel",),
        ),
    )(q, k, v)
    return out.reshape(B, H, S, D)
